# baseline (speedup 1.0000x reference)
import functools

import jax
import jax.numpy as jnp
import numpy as np
from jax import lax
from jax.experimental import pallas as pl
from jax.experimental.pallas import tpu as pltpu
from jax.experimental.pallas import tpu_sc as plsc

N = 100000
D = 256
G = 1024
NW = 32
ROWS_PER_W = N // NW
BP = 64
NBLK = 50
LOAD_PER_W = NBLK * BP
GPAD = G + 8
CL = 16
STRIPE = G // 16


_SC_MESH = plsc.VectorSubcoreMesh(core_axis_name="c", subcore_axis_name="s")


@functools.partial(
    pl.kernel,
    mesh=_SC_MESH,
    out_type=[
        jax.ShapeDtypeStruct((2 * G, D), jnp.float32),
        jax.ShapeDtypeStruct((2 * G, CL), jnp.float32),
    ],
    scratch_types=[
        pltpu.VMEM((NBLK, BP), jnp.int32),
        pltpu.VMEM((BP, D), jnp.float32),
        pltpu.VMEM((BP, D), jnp.float32),
        pltpu.VMEM((BP, D), jnp.float32),
        pltpu.VMEM((BP, D), jnp.float32),
        pltpu.VMEM((BP, D), jnp.float32),
        pltpu.VMEM((BP, CL), jnp.float32),
        pltpu.VMEM_SHARED((GPAD, D), jnp.float32),
        pltpu.VMEM_SHARED((GPAD, CL), jnp.float32),
        pltpu.SemaphoreType.DMA,
        pltpu.SemaphoreType.DMA,
        pltpu.SemaphoreType.DMA,
        pltpu.SemaphoreType.DMA,
        pltpu.SemaphoreType.DMA,
        pltpu.SemaphoreType.DMA,
        pltpu.SemaphoreType.DMA,
        pltpu.SemaphoreType.DMA,
        pltpu.SemaphoreType.DMA,
        pltpu.SemaphoreType.DMA,
        pltpu.SemaphoreType.DMA,
    ],
    compiler_params=pltpu.CompilerParams(use_tc_tiling_on_sc=False),
)
def _seg_sum_sc(ids_hbm, h_hbm, zsum_hbm, zcnt_hbm, ones_hbm,
                sums_hbm, cnts_hbm,
                ids_v, buf0, buf1, buf2, buf3, buf4, ones_v, acc_s, cnt_s,
                ld0, ld1, ld2, ld3, ld4, st0, st1, st2, st3, st4, ctsem):
    cid = lax.axis_index("c")
    sid = lax.axis_index("s")
    wid = sid * 2 + cid
    bufs = (buf0, buf1, buf2, buf3, buf4)
    lds = (ld0, ld1, ld2, ld3, ld4)
    sts = (st0, st1, st2, st3, st4)

    pltpu.sync_copy(ids_hbm.at[wid], ids_v)
    pltpu.sync_copy(ones_hbm, ones_v)
    pltpu.sync_copy(zsum_hbm.at[pl.ds(sid * STRIPE, STRIPE)],
                    acc_s.at[pl.ds(sid * STRIPE, STRIPE)])
    pltpu.sync_copy(zcnt_hbm.at[pl.ds(sid * STRIPE, STRIPE)],
                    cnt_s.at[pl.ds(sid * STRIPE, STRIPE)])
    plsc.subcore_barrier()

    row0 = jnp.minimum(wid * ROWS_PER_W // 8 * 8, N - LOAD_PER_W)

    def h_src(b):
        return h_hbm.at[pl.ds(row0 + b * BP, BP)]

    def start_scat(b, k):
        pltpu.async_copy(bufs[k], acc_s.at[ids_v.at[b]], sts[k], add=True)

    def wait_scat(b, k):
        pltpu.make_async_copy(bufs[k], acc_s.at[ids_v.at[b]], sts[k]).wait()

    def cnt_fire(b, carry):
        pltpu.async_copy(ones_v, cnt_s.at[ids_v.at[b]], ctsem, add=True)
        return carry

    lax.fori_loop(0, NBLK, cnt_fire, 0)

    pltpu.async_copy(h_src(0), bufs[0], lds[0])
    pltpu.async_copy(h_src(1), bufs[1], lds[1])

    def group(g, carry):
        for k in range(5):
            b = g * 5 + k
            kn = (k + 2) % 5
            @pl.when(b >= 3)
            def _():
                wait_scat(b - 3, kn)
            @pl.when(b + 2 < NBLK)
            def _():
                pltpu.async_copy(h_src(b + 2), bufs[kn], lds[kn])
            pltpu.make_async_copy(h_src(b), bufs[k], lds[k]).wait()
            start_scat(b, k)
        return carry

    assert NBLK % 5 == 0
    lax.fori_loop(0, NBLK // 5, group, 0)

    wait_scat(NBLK - 3, (NBLK - 3) % 5)
    wait_scat(NBLK - 2, (NBLK - 2) % 5)
    wait_scat(NBLK - 1, (NBLK - 1) % 5)

    def cnt_drain(b, carry):
        pltpu.make_async_copy(ones_v, cnt_s.at[ids_v.at[b]], ctsem).wait()
        return carry

    lax.fori_loop(0, NBLK, cnt_drain, 0)
    plsc.subcore_barrier()

    pltpu.sync_copy(acc_s.at[pl.ds(sid * STRIPE, STRIPE)],
                    sums_hbm.at[pl.ds(cid * G + sid * STRIPE, STRIPE)])
    pltpu.sync_copy(cnt_s.at[pl.ds(sid * STRIPE, STRIPE)],
                    cnts_hbm.at[pl.ds(cid * G + sid * STRIPE, STRIPE)])


def _mlp_body(sums_ref, cnts_ref, fcw_ref, fcb_ref, clsw_ref, clsb_ref,
              out_ref):
    sums = sums_ref[:G] + sums_ref[G:]
    cnt = cnts_ref[:G] + cnts_ref[G:]
    cnt0 = jnp.maximum(cnt[:, 0:1], 1.0)
    gf = sums / cnt0
    hidden = jnp.maximum(jnp.dot(gf, fcw_ref[...]) + fcb_ref[...], 0.0)
    out_ref[...] = jnp.dot(hidden, clsw_ref[...]) + clsb_ref[...]


_STARTS = [min(w * ROWS_PER_W // 8 * 8, N - LOAD_PER_W) for w in range(NW)]
_OWNED = np.stack([
    (np.arange(s, s + LOAD_PER_W) >= w * ROWS_PER_W)
    & (np.arange(s, s + LOAD_PER_W) < (w + 1) * ROWS_PER_W)
    for w, s in enumerate(_STARTS)
])


def _build_ids(graph_ids):
    gid = graph_ids.astype(jnp.int32)
    wins = jnp.stack([lax.slice(gid, (s,), (s + LOAD_PER_W,))
                      for s in _STARTS])
    ids = jnp.where(_OWNED, wins, G)
    return ids.reshape(NW, NBLK, BP)


def kernel(h, graph_ids, fc_w, fc_b, cls_w, cls_b):
    ids = _build_ids(graph_ids)
    zsum = jnp.zeros((G, D), jnp.float32)
    zcnt = jnp.zeros((G, CL), jnp.float32)
    ones = jnp.ones((BP, CL), jnp.float32)

    sums2, cnts2 = _seg_sum_sc(ids, h, zsum, zcnt, ones)

    out = pl.pallas_call(
        _mlp_body,
        out_shape=jax.ShapeDtypeStruct((G, 16), jnp.float32),
    )(sums2, cnts2,
      fc_w, fc_b.reshape(1, 512), cls_w, cls_b.reshape(1, 16))
    return out
